# Initial kernel scaffold; baseline (speedup 1.0000x reference)
import jax
import jax.numpy as jnp
from jax.experimental import pallas as pl


def kernel(adj_indices, adj_values, user_emb_w, item_emb_w):
    raise NotImplementedError("write your pallas kernel here")


# speedup vs baseline: 1.8750x; 1.8750x over previous
import jax
import jax.numpy as jnp
from jax import lax
from jax.experimental import pallas as pl
from jax.experimental.pallas import tpu as pltpu
from jax.experimental.pallas import tpu_sc as plsc

_NUM_USER = 30000
_NUM_ITEM = 20000
_N = _NUM_USER + _NUM_ITEM
_G = 16
_E = 800000
_C = 128
_PER_TILE = 50048
_NCH = _PER_TILE // _C
_E_PAD = 16 * _PER_TILE
_RPT = _N // 16
_RC = 125
_NRC = _RPT // _RC


def _sc_body(row_ref, col_ref, val_ref, x0_ref, out_ref, xs1, xs2,
             acc, maccum, gbuf, wbuf, zbuf, colb, rowb, valb):
    c = lax.axis_index("c")
    s = lax.axis_index("s")
    r0 = s * _RPT
    base0 = s * _PER_TILE
    zero16 = jnp.zeros((16,), jnp.float32)

    @pl.loop(0, _RC)
    def _(i):
        zbuf[i] = zero16

    def _edges(src, soff):
        @pl.loop(0, _NCH)
        def _(j):
            b = base0 + j * _C
            pltpu.sync_copy(row_ref.at[pl.ds(b, _C)], rowb)
            pltpu.sync_copy(col_ref.at[pl.ds(b, _C)], colb)
            pltpu.sync_copy(val_ref.at[pl.ds(b, _C)], valb)

            off16 = jnp.full((16,), soff, jnp.int32)

            @pl.loop(0, _C // 16)
            def _(k):
                colb[pl.ds(k * 16, 16)] = colb[pl.ds(k * 16, 16)] + off16

            pltpu.sync_copy(src.at[colb], gbuf)

            @pl.loop(0, _C // 16)
            def _(g):
                vals = valb[pl.ds(g * 16, 16)]
                for t in range(16):
                    e = g * 16 + t
                    gbuf[e] = gbuf[e] * vals[t]

            pltpu.sync_copy(gbuf, acc.at[rowb], add=True)

        plsc.subcore_barrier()

    def _run_group(goff):
        xoff = c * jnp.int32(_N)

        @pl.loop(0, _NRC)
        def _(k):
            r = r0 + k * _RC
            pltpu.sync_copy(x0_ref.at[pl.ds(goff + r, _RC)], wbuf)
            pltpu.sync_copy(zbuf, acc.at[pl.ds(r, _RC)])

            @pl.loop(0, _RC, unroll=8)
            def _(i):
                maccum[k * _RC + i] = wbuf[i]

        plsc.subcore_barrier()

        def _writeout(dst_xs):
            @pl.loop(0, _NRC)
            def _(k):
                r = r0 + k * _RC
                pltpu.sync_copy(acc.at[pl.ds(r, _RC)], wbuf)
                pltpu.sync_copy(wbuf, dst_xs.at[pl.ds(xoff + r, _RC)])
                pltpu.sync_copy(zbuf, acc.at[pl.ds(r, _RC)])

                @pl.loop(0, _RC, unroll=8)
                def _(i):
                    m = k * _RC + i
                    maccum[m] = maccum[m] + wbuf[i]

            plsc.subcore_barrier()

        _edges(x0_ref, goff)
        _writeout(xs1)
        _edges(xs1, xoff)
        _writeout(xs2)
        _edges(xs2, xoff)

        @pl.loop(0, _NRC)
        def _(k):
            r = r0 + k * _RC
            pltpu.sync_copy(acc.at[pl.ds(r, _RC)], wbuf)

            @pl.loop(0, _RC, unroll=8)
            def _(i):
                m = k * _RC + i
                wbuf[i] = (maccum[m] + wbuf[i]) * 0.25

            pltpu.sync_copy(wbuf, out_ref.at[pl.ds(goff + r, _RC)])

        plsc.subcore_barrier()

    _run_group(c * jnp.int32(_N))
    _run_group((2 + c) * jnp.int32(_N))


@jax.jit
def _sc_call(row_p, col_p, val_p, x0cat):
    mesh = plsc.VectorSubcoreMesh(core_axis_name="c", subcore_axis_name="s")
    f = pl.kernel(
        _sc_body,
        out_type=jax.ShapeDtypeStruct((4 * _N, _G), jnp.float32),
        mesh=mesh,
        compiler_params=pltpu.CompilerParams(use_tc_tiling_on_sc=False),
        scratch_types=[
            pltpu.HBM((2 * _N, _G), jnp.float32),
            pltpu.HBM((2 * _N, _G), jnp.float32),
            pltpu.VMEM_SHARED((_N, _G), jnp.float32),
            pltpu.VMEM((_RPT, _G), jnp.float32),
            pltpu.VMEM((_C, _G), jnp.float32),
            pltpu.VMEM((_RC, _G), jnp.float32),
            pltpu.VMEM((_RC, _G), jnp.float32),
            pltpu.VMEM((_C,), jnp.int32),
            pltpu.VMEM((_C,), jnp.int32),
            pltpu.VMEM((_C,), jnp.float32),
        ],
    )
    return f(row_p, col_p, val_p, x0cat)


def kernel(adj_indices, adj_values, user_emb_w, item_emb_w):
    x0 = jnp.concatenate([user_emb_w, item_emb_w], axis=0)
    x0cat = jnp.concatenate([x0[:, g * _G:(g + 1) * _G] for g in range(4)],
                            axis=0)
    pad = _E_PAD - _E
    pidx = jnp.arange(pad, dtype=jnp.int32) % _N
    row_p = jnp.concatenate([adj_indices[0].astype(jnp.int32), pidx])
    col_p = jnp.concatenate([adj_indices[1].astype(jnp.int32), pidx])
    val_p = jnp.concatenate([adj_values, jnp.zeros((pad,), jnp.float32)])
    outcat = _sc_call(row_p, col_p, val_p, x0cat)
    mean = jnp.concatenate([outcat[g * _N:(g + 1) * _N] for g in range(4)],
                           axis=1)
    return mean[:_NUM_USER], mean[_NUM_USER:]


# speedup vs baseline: 5.1990x; 2.7729x over previous
import jax
import jax.numpy as jnp
from jax import lax
from jax.experimental import pallas as pl
from jax.experimental.pallas import tpu as pltpu
from jax.experimental.pallas import tpu_sc as plsc

_NUM_USER = 30000
_NUM_ITEM = 20000
_N = _NUM_USER + _NUM_ITEM
_G = 16
_E = 800000
_C = 128
_BLK = 2048
_NBLK = 25
_PER_TILE = _NBLK * _BLK
_E_PAD = 16 * _PER_TILE
_RPT = _N // 16
_RC = 125
_NRC = _RPT // _RC


def _sc_body(row_ref, col_ref, val_ref, x0_ref, out_ref, xs1, xs2,
             acc, maccum, g0, g1, wbuf, zbuf, colb, rowb, valb,
             sg0, sg1):
    c = lax.axis_index("c")
    s = lax.axis_index("s")
    r0 = s * _RPT
    blk0 = s * _NBLK
    zero16 = jnp.zeros((16,), jnp.float32)

    @pl.loop(0, _RC)
    def _(i):
        zbuf[i] = zero16

    def _scale(gb, v0):
        @pl.loop(0, _C // 16)
        def _(g):
            vals = valb[pl.ds(v0 + g * 16, 16)]
            for t in range(16):
                e = g * 16 + t
                gb[e] = gb[e] * vals[t]

    def _edges(src, soff):
        off16 = jnp.full((16,), soff, jnp.int32)

        @pl.loop(0, _NBLK)
        def _(b):
            base = (blk0 + b) * _BLK
            pltpu.sync_copy(col_ref.at[pl.ds(base, _BLK)], colb)
            pltpu.sync_copy(val_ref.at[pl.ds(base, _BLK)], valb)
            pltpu.sync_copy(row_ref.at[pl.ds((blk0 + b) * (_BLK // _C),
                                             _BLK // _C)], rowb)

            @pl.loop(0, _BLK // 16)
            def _(m):
                colb[pl.ds(m * 16, 16)] = colb[pl.ds(m * 16, 16)] + off16

            pltpu.make_async_copy(
                src.at[colb.at[pl.ds(0, _C)]], g0, sg0).start()

            @pl.loop(0, _BLK // (2 * _C))
            def _(p):
                k0 = 2 * p
                pltpu.make_async_copy(
                    src.at[colb.at[pl.ds((k0 + 1) * _C, _C)]], g1,
                    sg1).start()
                pltpu.make_async_copy(
                    src.at[colb.at[pl.ds(k0 * _C, _C)]], g0, sg0).wait()
                _scale(g0, k0 * _C)
                pltpu.sync_copy(g0, acc.at[rowb.at[k0]], add=True)

                @pl.when(p < _BLK // (2 * _C) - 1)
                def _():
                    pltpu.make_async_copy(
                        src.at[colb.at[pl.ds((k0 + 2) * _C, _C)]], g0,
                        sg0).start()

                pltpu.make_async_copy(
                    src.at[colb.at[pl.ds((k0 + 1) * _C, _C)]], g1,
                    sg1).wait()
                _scale(g1, (k0 + 1) * _C)
                pltpu.sync_copy(g1, acc.at[rowb.at[k0 + 1]], add=True)

        plsc.subcore_barrier()

    def _run_group(goff):
        xoff = c * jnp.int32(_N)

        @pl.loop(0, _NRC)
        def _(k):
            r = r0 + k * _RC
            pltpu.sync_copy(x0_ref.at[pl.ds(goff + r, _RC)], wbuf)
            pltpu.sync_copy(zbuf, acc.at[pl.ds(r, _RC)])

            @pl.loop(0, _RC, unroll=8)
            def _(i):
                maccum[k * _RC + i] = wbuf[i]

        plsc.subcore_barrier()

        def _writeout(dst_xs):
            @pl.loop(0, _NRC)
            def _(k):
                r = r0 + k * _RC
                pltpu.sync_copy(acc.at[pl.ds(r, _RC)], wbuf)
                pltpu.sync_copy(wbuf, dst_xs.at[pl.ds(xoff + r, _RC)])
                pltpu.sync_copy(zbuf, acc.at[pl.ds(r, _RC)])

                @pl.loop(0, _RC, unroll=8)
                def _(i):
                    m = k * _RC + i
                    maccum[m] = maccum[m] + wbuf[i]

            plsc.subcore_barrier()

        _edges(x0_ref, goff)
        _writeout(xs1)
        _edges(xs1, xoff)
        _writeout(xs2)
        _edges(xs2, xoff)

        @pl.loop(0, _NRC)
        def _(k):
            r = r0 + k * _RC
            pltpu.sync_copy(acc.at[pl.ds(r, _RC)], wbuf)

            @pl.loop(0, _RC, unroll=8)
            def _(i):
                m = k * _RC + i
                wbuf[i] = (maccum[m] + wbuf[i]) * 0.25

            pltpu.sync_copy(wbuf, out_ref.at[pl.ds(goff + r, _RC)])

        plsc.subcore_barrier()

    _run_group(c * jnp.int32(_N))
    _run_group((2 + c) * jnp.int32(_N))


@jax.jit
def _sc_call(row_p, col_p, val_p, x0cat):
    mesh = plsc.VectorSubcoreMesh(core_axis_name="c", subcore_axis_name="s")
    f = pl.kernel(
        _sc_body,
        out_type=jax.ShapeDtypeStruct((4 * _N, _G), jnp.float32),
        mesh=mesh,
        compiler_params=pltpu.CompilerParams(use_tc_tiling_on_sc=False),
        scratch_types=[
            pltpu.HBM((2 * _N, _G), jnp.float32),
            pltpu.HBM((2 * _N, _G), jnp.float32),
            pltpu.VMEM_SHARED((_N, _G), jnp.float32),
            pltpu.VMEM((_RPT, _G), jnp.float32),
            pltpu.VMEM((_C, _G), jnp.float32),
            pltpu.VMEM((_C, _G), jnp.float32),
            pltpu.VMEM((_RC, _G), jnp.float32),
            pltpu.VMEM((_RC, _G), jnp.float32),
            pltpu.VMEM((_BLK,), jnp.int32),
            pltpu.VMEM((_BLK // _C, _C), jnp.int32),
            pltpu.VMEM((_BLK,), jnp.float32),
            pltpu.SemaphoreType.DMA,
            pltpu.SemaphoreType.DMA,
        ],
    )
    return f(row_p, col_p, val_p, x0cat)


def kernel(adj_indices, adj_values, user_emb_w, item_emb_w):
    x0 = jnp.concatenate([user_emb_w, item_emb_w], axis=0)
    x0cat = jnp.concatenate([x0[:, g * _G:(g + 1) * _G] for g in range(4)],
                            axis=0)
    pad = _E_PAD - _E
    pidx = jnp.arange(pad, dtype=jnp.int32) % _N
    row_p = jnp.concatenate([adj_indices[0].astype(jnp.int32), pidx])
    col_p = jnp.concatenate([adj_indices[1].astype(jnp.int32), pidx])
    val_p = jnp.concatenate([adj_values, jnp.zeros((pad,), jnp.float32)])
    row2d = row_p.reshape(_E_PAD // _C, _C)
    outcat = _sc_call(row2d, col_p, val_p, x0cat)
    mean = jnp.concatenate([outcat[g * _N:(g + 1) * _N] for g in range(4)],
                           axis=1)
    return mean[:_NUM_USER], mean[_NUM_USER:]


# speedup vs baseline: 6.3421x; 1.2199x over previous
import jax
import jax.numpy as jnp
from jax import lax
from jax.experimental import pallas as pl
from jax.experimental.pallas import tpu as pltpu
from jax.experimental.pallas import tpu_sc as plsc

_NUM_USER = 30000
_NUM_ITEM = 20000
_N = _NUM_USER + _NUM_ITEM
_G = 16
_E = 800000
_C = 128
_BLK = 1024
_NCPB = _BLK // _C
_NBLK = 50
_PER_TILE = _NBLK * _BLK
_E_PAD = 16 * _PER_TILE
_RPT = _N // 16
_RC = 125
_NRC = _RPT // _RC


def _sc_body(row_ref, col_ref, val_ref, x0_ref, out_ref, xt,
             acc, maccum, g0, g1, g2, wbuf, zbuf,
             colb0, rowb0, valb0, colb1, rowb1, valb1,
             sg0, sg1, sg2, ss0, ss1, ss2, si0, si1):
    c = lax.axis_index("c")
    s = lax.axis_index("s")
    r0 = s * _RPT
    blk0 = s * _NBLK
    zero16 = jnp.zeros((16,), jnp.float32)

    gbufs = (g0, g1, g2)
    gsems = (sg0, sg1, sg2)
    ssems = (ss0, ss1, ss2)
    idxbufs = ((colb0, rowb0, valb0, si0), (colb1, rowb1, valb1, si1))

    @pl.loop(0, _RC)
    def _(i):
        zbuf[i] = zero16

    def _fire_idx(b, buf):
        cb, rb, vb, sem = buf
        base = (blk0 + b) * _BLK
        pltpu.make_async_copy(col_ref.at[pl.ds(base, _BLK)], cb, sem).start()
        pltpu.make_async_copy(val_ref.at[pl.ds(base, _BLK)], vb, sem).start()
        pltpu.make_async_copy(
            row_ref.at[pl.ds((blk0 + b) * _NCPB, _NCPB)], rb, sem).start()

    def _wait_idx(buf):
        cb, rb, vb, sem = buf
        pltpu.make_async_copy(col_ref.at[pl.ds(0, _BLK)], cb, sem).wait()
        pltpu.make_async_copy(val_ref.at[pl.ds(0, _BLK)], vb, sem).wait()
        pltpu.make_async_copy(row_ref.at[pl.ds(0, _NCPB)], rb, sem).wait()

    def _scale(gb, vb, v0):
        @pl.loop(0, _C // 16)
        def _(g):
            vals = vb[pl.ds(v0 + g * 16, 16)]
            for t in range(16):
                e = g * 16 + t
                gb[e] = gb[e] * vals[t]

    def _edges(src, soff):
        off16 = jnp.full((16,), soff, jnp.int32)

        _fire_idx(0, idxbufs[0])

        def _block(b, buf, nxt):
            cb, rb, vb, _ = buf
            _wait_idx(buf)

            @pl.when(b + 1 < _NBLK)
            def _():
                _fire_idx(b + 1, nxt)

            @pl.loop(0, _BLK // 16)
            def _(m):
                cb[pl.ds(m * 16, 16)] = cb[pl.ds(m * 16, 16)] + off16

            gcp = {}
            scp = {}

            def fire_gather(k):
                sl = k % 3
                cp = pltpu.make_async_copy(
                    src.at[cb.at[pl.ds(k * _C, _C)]], gbufs[sl], gsems[sl])
                cp.start()
                gcp[k] = cp

            def fire_scatter(k):
                sl = k % 3
                cp = pltpu.make_async_copy(
                    gbufs[sl], acc.at[rb.at[k]], ssems[sl])
                cp.start(add=True)
                scp[k] = cp

            fire_gather(0)
            fire_gather(1)
            for k in range(_NCPB):
                gcp[k].wait()
                _scale(gbufs[k % 3], vb, k * _C)
                fire_scatter(k)
                if k + 2 < _NCPB:
                    if k >= 1:
                        scp[k - 1].wait()
                    fire_gather(k + 2)
            for k in range(_NCPB - 3, _NCPB):
                scp[k].wait()

        @pl.loop(0, _NBLK, step=2)
        def _(b):
            _block(b, idxbufs[0], idxbufs[1])
            _block(b + 1, idxbufs[1], idxbufs[0])

        plsc.subcore_barrier()

    def _writeout(goff_unused, xoff):
        @pl.loop(0, _NRC)
        def _(k):
            r = r0 + k * _RC
            pltpu.sync_copy(acc.at[pl.ds(r, _RC)], wbuf)
            pltpu.sync_copy(wbuf, xt.at[pl.ds(xoff + r, _RC)])
            pltpu.sync_copy(zbuf, acc.at[pl.ds(r, _RC)])

            @pl.loop(0, _RC, unroll=8)
            def _(i):
                m = k * _RC + i
                maccum[m] = maccum[m] + wbuf[i]

        plsc.subcore_barrier()

    @pl.loop(0, 2)
    def _(ph):
        goff = (2 * ph + c) * jnp.int32(_N)
        xoff = c * jnp.int32(_N)

        @pl.loop(0, _NRC)
        def _(k):
            r = r0 + k * _RC
            pltpu.sync_copy(x0_ref.at[pl.ds(goff + r, _RC)], wbuf)
            pltpu.sync_copy(zbuf, acc.at[pl.ds(r, _RC)])

            @pl.loop(0, _RC, unroll=8)
            def _(i):
                maccum[k * _RC + i] = wbuf[i]

        plsc.subcore_barrier()

        _edges(x0_ref, goff)

        @pl.loop(0, 2)
        def _(l):
            _writeout(goff, xoff)
            _edges(xt, xoff)

        @pl.loop(0, _NRC)
        def _(k):
            r = r0 + k * _RC
            pltpu.sync_copy(acc.at[pl.ds(r, _RC)], wbuf)

            @pl.loop(0, _RC, unroll=8)
            def _(i):
                m = k * _RC + i
                wbuf[i] = (maccum[m] + wbuf[i]) * 0.25

            pltpu.sync_copy(wbuf, out_ref.at[pl.ds(goff + r, _RC)])

        plsc.subcore_barrier()


@jax.jit
def _sc_call(row_p, col_p, val_p, x0cat):
    mesh = plsc.VectorSubcoreMesh(core_axis_name="c", subcore_axis_name="s")
    f = pl.kernel(
        _sc_body,
        out_type=jax.ShapeDtypeStruct((4 * _N, _G), jnp.float32),
        mesh=mesh,
        compiler_params=pltpu.CompilerParams(use_tc_tiling_on_sc=False),
        scratch_types=[
            pltpu.HBM((2 * _N, _G), jnp.float32),
            pltpu.VMEM_SHARED((_N, _G), jnp.float32),
            pltpu.VMEM((_RPT, _G), jnp.float32),
            pltpu.VMEM((_C, _G), jnp.float32),
            pltpu.VMEM((_C, _G), jnp.float32),
            pltpu.VMEM((_C, _G), jnp.float32),
            pltpu.VMEM((_RC, _G), jnp.float32),
            pltpu.VMEM((_RC, _G), jnp.float32),
            pltpu.VMEM((_BLK,), jnp.int32),
            pltpu.VMEM((_NCPB, _C), jnp.int32),
            pltpu.VMEM((_BLK,), jnp.float32),
            pltpu.VMEM((_BLK,), jnp.int32),
            pltpu.VMEM((_NCPB, _C), jnp.int32),
            pltpu.VMEM((_BLK,), jnp.float32),
            pltpu.SemaphoreType.DMA,
            pltpu.SemaphoreType.DMA,
            pltpu.SemaphoreType.DMA,
            pltpu.SemaphoreType.DMA,
            pltpu.SemaphoreType.DMA,
            pltpu.SemaphoreType.DMA,
            pltpu.SemaphoreType.DMA,
            pltpu.SemaphoreType.DMA,
        ],
    )
    return f(row_p, col_p, val_p, x0cat)


def kernel(adj_indices, adj_values, user_emb_w, item_emb_w):
    x0 = jnp.concatenate([user_emb_w, item_emb_w], axis=0)
    x0cat = jnp.concatenate([x0[:, g * _G:(g + 1) * _G] for g in range(4)],
                            axis=0)
    pad = _E_PAD - _E
    pidx = jnp.arange(pad, dtype=jnp.int32) % _N
    row_p = jnp.concatenate([adj_indices[0].astype(jnp.int32), pidx])
    col_p = jnp.concatenate([adj_indices[1].astype(jnp.int32), pidx])
    val_p = jnp.concatenate([adj_values, jnp.zeros((pad,), jnp.float32)])
    row2d = row_p.reshape(_E_PAD // _C, _C)
    outcat = _sc_call(row2d, col_p, val_p, x0cat)
    mean = jnp.concatenate([outcat[g * _N:(g + 1) * _N] for g in range(4)],
                           axis=1)
    return mean[:_NUM_USER], mean[_NUM_USER:]


# speedup vs baseline: 7.2912x; 1.1497x over previous
import jax
import jax.numpy as jnp
from jax import lax
from jax.experimental import pallas as pl
from jax.experimental.pallas import tpu as pltpu
from jax.experimental.pallas import tpu_sc as plsc

_NUM_USER = 30000
_NUM_ITEM = 20000
_N = _NUM_USER + _NUM_ITEM
_NP = 51200
_G = 16
_E = 800000
_C = 128
_BLK = 1024
_NCPB = _BLK // _C
_NBLK = 50
_PER_TILE = _NBLK * _BLK
_E_PAD = 16 * _PER_TILE
_RPT = _NP // 16
_RC = 128
_NRC = _RPT // _RC


def _sc_body(row_ref, col_ref, val_ref, x0_ref, out_ref, xt,
             acc, maccum, g0, g1, g2, g3, wbuf, zbuf, sidx,
             colb0, rowb0, valb0, colb1, rowb1, valb1,
             sg0, sg1, sg2, sg3, ss0, ss1, ss2, ss3, si0, si1):
    c = lax.axis_index("c")
    s = lax.axis_index("s")
    r0 = s * _RPT
    blk0 = s * _NBLK
    zero16 = jnp.zeros((16,), jnp.float32)
    lane = jnp.arange(16, dtype=jnp.int32)

    gbufs = (g0, g1, g2, g3)
    gsems = (sg0, sg1, sg2, sg3)
    ssems = (ss0, ss1, ss2, ss3)
    idxbufs = ((colb0, rowb0, valb0, si0), (colb1, rowb1, valb1, si1))

    @pl.loop(0, _RC)
    def _(i):
        zbuf[i] = zero16

    def _fire_idx(b, buf):
        cb, rb, vb, sem = buf
        base = (blk0 + b) * _BLK
        pltpu.make_async_copy(col_ref.at[pl.ds(base, _BLK)], cb, sem).start()
        pltpu.make_async_copy(val_ref.at[pl.ds(base, _BLK)], vb, sem).start()
        pltpu.make_async_copy(
            row_ref.at[pl.ds((blk0 + b) * _NCPB, _NCPB)], rb, sem).start()

    def _wait_idx(buf):
        cb, rb, vb, sem = buf
        pltpu.make_async_copy(col_ref.at[pl.ds(0, _BLK)], cb, sem).wait()
        pltpu.make_async_copy(val_ref.at[pl.ds(0, _BLK)], vb, sem).wait()
        pltpu.make_async_copy(row_ref.at[pl.ds(0, _NCPB)], rb, sem).wait()

    def _scale(gb, vb, v0):
        @pl.loop(0, _C // 16)
        def _(g):
            vals = vb[pl.ds(v0 + g * 16, 16)]
            for t in range(16):
                e = g * 16 + t
                gb[e] = gb[e] * vals[t]

    def _edges(src, mul, off):
        off16 = jnp.full((16,), off, jnp.int32)

        _fire_idx(0, idxbufs[0])

        def _block(b, buf, nxt):
            cb, rb, vb, _ = buf
            _wait_idx(buf)

            @pl.when(b + 1 < _NBLK)
            def _():
                _fire_idx(b + 1, nxt)

            @pl.loop(0, _BLK // 16)
            def _(m):
                cb[pl.ds(m * 16, 16)] = cb[pl.ds(m * 16, 16)] * mul + off16

            gcp = {}
            scp = {}

            def fire_gather(k):
                sl = k % 4
                cp = pltpu.make_async_copy(
                    src.at[cb.at[pl.ds(k * _C, _C)]], gbufs[sl], gsems[sl])
                cp.start()
                gcp[k] = cp

            def fire_scatter(k):
                sl = k % 4
                cp = pltpu.make_async_copy(
                    gbufs[sl], acc.at[rb.at[k]], ssems[sl])
                cp.start(add=True)
                scp[k] = cp

            fire_gather(0)
            fire_gather(1)
            fire_gather(2)
            for k in range(_NCPB):
                gcp[k].wait()
                _scale(gbufs[k % 4], vb, k * _C)
                fire_scatter(k)
                if k + 3 < _NCPB:
                    if k >= 1:
                        scp[k - 1].wait()
                    fire_gather(k + 3)
            for k in range(_NCPB - 4, _NCPB):
                scp[k].wait()

        @pl.loop(0, _NBLK, step=2)
        def _(b):
            _block(b, idxbufs[0], idxbufs[1])
            _block(b + 1, idxbufs[1], idxbufs[0])

        plsc.subcore_barrier()

    def _writeout(xoff):
        @pl.loop(0, _NRC)
        def _(k):
            r = r0 + k * _RC
            pltpu.sync_copy(acc.at[pl.ds(r, _RC)], wbuf)
            pltpu.sync_copy(wbuf, xt.at[pl.ds(xoff + r, _RC)])
            pltpu.sync_copy(zbuf, acc.at[pl.ds(r, _RC)])

            @pl.loop(0, _RC, unroll=8)
            def _(i):
                m = k * _RC + i
                maccum[m] = maccum[m] + wbuf[i]

        plsc.subcore_barrier()

    @pl.loop(0, 2)
    def _(ph):
        g = 2 * ph + c
        xoff = c * jnp.int32(_NP)

        @pl.loop(0, _NRC)
        def _(k):
            @pl.loop(0, _RC // 16)
            def _(m):
                node = lane + (r0 + k * _RC + m * 16)
                idx = jnp.minimum(node, _N - 1) * 4 + g
                sidx[pl.ds(m * 16, 16)] = idx

            pltpu.sync_copy(x0_ref.at[sidx],
                            maccum.at[pl.ds(k * _RC, _RC)])
            pltpu.sync_copy(zbuf, acc.at[pl.ds(r0 + k * _RC, _RC)])

        plsc.subcore_barrier()

        _edges(x0_ref, jnp.int32(4), g)

        @pl.loop(0, 2)
        def _(l):
            _writeout(xoff)
            _edges(xt, jnp.int32(1), xoff)

        @pl.loop(0, _NRC)
        def _(k):
            r = r0 + k * _RC
            pltpu.sync_copy(acc.at[pl.ds(r, _RC)], wbuf)

            @pl.loop(0, _RC, unroll=8)
            def _(i):
                m = k * _RC + i
                wbuf[i] = (maccum[m] + wbuf[i]) * 0.25

            @pl.loop(0, _RC // 16)
            def _(m):
                node = lane + (r + m * 16)
                sidx[pl.ds(m * 16, 16)] = node * 4 + g

            pltpu.sync_copy(wbuf, out_ref.at[sidx])

        plsc.subcore_barrier()


@jax.jit
def _sc_call(row_p, col_p, val_p, x0il):
    mesh = plsc.VectorSubcoreMesh(core_axis_name="c", subcore_axis_name="s")
    f = pl.kernel(
        _sc_body,
        out_type=jax.ShapeDtypeStruct((4 * _NP, _G), jnp.float32),
        mesh=mesh,
        compiler_params=pltpu.CompilerParams(use_tc_tiling_on_sc=False),
        scratch_types=[
            pltpu.HBM((2 * _NP, _G), jnp.float32),
            pltpu.VMEM_SHARED((_NP, _G), jnp.float32),
            pltpu.VMEM((_RPT, _G), jnp.float32),
            pltpu.VMEM((_C, _G), jnp.float32),
            pltpu.VMEM((_C, _G), jnp.float32),
            pltpu.VMEM((_C, _G), jnp.float32),
            pltpu.VMEM((_C, _G), jnp.float32),
            pltpu.VMEM((_RC, _G), jnp.float32),
            pltpu.VMEM((_RC, _G), jnp.float32),
            pltpu.VMEM((_C,), jnp.int32),
            pltpu.VMEM((_BLK,), jnp.int32),
            pltpu.VMEM((_NCPB, _C), jnp.int32),
            pltpu.VMEM((_BLK,), jnp.float32),
            pltpu.VMEM((_BLK,), jnp.int32),
            pltpu.VMEM((_NCPB, _C), jnp.int32),
            pltpu.VMEM((_BLK,), jnp.float32),
            pltpu.SemaphoreType.DMA,
            pltpu.SemaphoreType.DMA,
            pltpu.SemaphoreType.DMA,
            pltpu.SemaphoreType.DMA,
            pltpu.SemaphoreType.DMA,
            pltpu.SemaphoreType.DMA,
            pltpu.SemaphoreType.DMA,
            pltpu.SemaphoreType.DMA,
            pltpu.SemaphoreType.DMA,
            pltpu.SemaphoreType.DMA,
        ],
    )
    return f(row_p, col_p, val_p, x0il)


def kernel(adj_indices, adj_values, user_emb_w, item_emb_w):
    x0 = jnp.concatenate([user_emb_w, item_emb_w], axis=0)
    x0il = x0.reshape(4 * _N, _G)
    pad = _E_PAD - _E
    pidx = jnp.arange(pad, dtype=jnp.int32) % _N
    row_p = jnp.concatenate([adj_indices[0].astype(jnp.int32), pidx])
    col_p = jnp.concatenate([adj_indices[1].astype(jnp.int32), pidx])
    val_p = jnp.concatenate([adj_values, jnp.zeros((pad,), jnp.float32)])
    row2d = row_p.reshape(_E_PAD // _C, _C)
    outil = _sc_call(row2d, col_p, val_p, x0il)
    mean = outil.reshape(_NP, 4 * _G)[:_N]
    return mean[:_NUM_USER], mean[_NUM_USER:]


# speedup vs baseline: 10.9977x; 1.5084x over previous
import jax
import jax.numpy as jnp
from jax import lax
from jax.experimental import pallas as pl
from jax.experimental.pallas import tpu as pltpu
from jax.experimental.pallas import tpu_sc as plsc

_NUM_USER = 30000
_NUM_ITEM = 20000
_N = _NUM_USER + _NUM_ITEM
_NP = 51200
_H = 32
_E = 800000
_C = 128
_BLK = 1024
_NCPB = _BLK // _C
_NBLK = 50
_PER_TILE = _NBLK * _BLK
_E_PAD = 16 * _PER_TILE
_RPT = _NP // 16
_RC = 128
_NRC = _RPT // _RC


def _sc_body(row_ref, col_ref, val_ref, x0_ref, out_ref, xt,
             acc, g0, g1, g2, g3, zbuf,
             colb0, rowb0, valb0, colb1, rowb1, valb1,
             sg0, sg1, sg2, sg3, ss0, ss1, ss2, ss3, si0, si1):
    c = lax.axis_index("c")
    s = lax.axis_index("s")
    r0 = s * _RPT
    blk0 = s * _NBLK
    xoff = c * jnp.int32(_NP)
    zero16 = jnp.zeros((16,), jnp.float32)
    lane = jnp.arange(16, dtype=jnp.int32)

    gbufs = (g0, g1, g2, g3)
    gsems = (sg0, sg1, sg2, sg3)
    ssems = (ss0, ss1, ss2, ss3)
    idxbufs = ((colb0, rowb0, valb0, si0), (colb1, rowb1, valb1, si1))

    @pl.loop(0, 32)
    def _(i):
        zbuf[i, pl.ds(0, 16)] = zero16
        zbuf[i, pl.ds(16, 16)] = zero16

    def _zero_acc(r):
        for j in range(4):
            pltpu.sync_copy(zbuf, acc.at[pl.ds(r + 32 * j, 32)])

    def _fire_idx(b, buf):
        cb, rb, vb, sem = buf
        base = (blk0 + b) * _BLK
        pltpu.make_async_copy(col_ref.at[pl.ds(base, _BLK)], cb, sem).start()
        pltpu.make_async_copy(val_ref.at[pl.ds(base, _BLK)], vb, sem).start()
        pltpu.make_async_copy(
            row_ref.at[pl.ds((blk0 + b) * _NCPB, _NCPB)], rb, sem).start()

    def _wait_idx(buf):
        cb, rb, vb, sem = buf
        pltpu.make_async_copy(col_ref.at[pl.ds(0, _BLK)], cb, sem).wait()
        pltpu.make_async_copy(val_ref.at[pl.ds(0, _BLK)], vb, sem).wait()
        pltpu.make_async_copy(row_ref.at[pl.ds(0, _NCPB)], rb, sem).wait()

    def _scale(gb, vb, v0):
        @pl.loop(0, _C // 16)
        def _(g):
            vals = vb[pl.ds(v0 + g * 16, 16)]
            for t in range(16):
                e = g * 16 + t
                gb[e, pl.ds(0, 16)] = gb[e, pl.ds(0, 16)] * vals[t]
                gb[e, pl.ds(16, 16)] = gb[e, pl.ds(16, 16)] * vals[t]

    def _edges():
        xoff16 = jnp.full((16,), xoff, jnp.int32)

        _fire_idx(0, idxbufs[0])

        def _block(b, buf, nxt):
            cb, rb, vb, _ = buf
            _wait_idx(buf)

            @pl.when(b + 1 < _NBLK)
            def _():
                _fire_idx(b + 1, nxt)

            @pl.loop(0, _BLK // 16)
            def _(m):
                cb[pl.ds(m * 16, 16)] = cb[pl.ds(m * 16, 16)] + xoff16

            gcp = {}
            scp = {}

            def fire_gather(k):
                sl = k % 4
                cp = pltpu.make_async_copy(
                    xt.at[cb.at[pl.ds(k * _C, _C)]], gbufs[sl], gsems[sl])
                cp.start()
                gcp[k] = cp

            def fire_scatter(k):
                sl = k % 4
                cp = pltpu.make_async_copy(
                    gbufs[sl], acc.at[rb.at[k]], ssems[sl])
                cp.start(add=True)
                scp[k] = cp

            fire_gather(0)
            fire_gather(1)
            fire_gather(2)
            for k in range(_NCPB):
                gcp[k].wait()
                _scale(gbufs[k % 4], vb, k * _C)
                fire_scatter(k)
                if k + 3 < _NCPB:
                    if k >= 1:
                        scp[k - 1].wait()
                    fire_gather(k + 3)
            for k in range(_NCPB - 4, _NCPB):
                scp[k].wait()

        @pl.loop(0, _NBLK, step=2)
        def _(b):
            _block(b, idxbufs[0], idxbufs[1])
            _block(b + 1, idxbufs[1], idxbufs[0])

        plsc.subcore_barrier()

    @pl.loop(0, _NRC)
    def _(k):
        r = r0 + k * _RC

        @pl.loop(0, _RC // 16)
        def _(m):
            node = lane + (r + m * 16)
            rowb0[0, pl.ds(m * 16, 16)] = jnp.minimum(node, _N - 1) * 2 + c

        pltpu.sync_copy(x0_ref.at[rowb0.at[0]], g0)
        pltpu.sync_copy(g0, xt.at[pl.ds(xoff + r, _RC)])
        pltpu.sync_copy(g0, out_ref.at[pl.ds(xoff + r, _RC)])
        _zero_acc(r)

    plsc.subcore_barrier()

    @pl.loop(0, 3)
    def _(l):
        _edges()

        lvec = jnp.full((16,), l, jnp.int32)
        sc16 = jnp.where(lvec == 2, jnp.float32(0.25), jnp.float32(1.0))

        @pl.loop(0, _NRC)
        def _(k):
            r = r0 + k * _RC
            pltpu.sync_copy(acc.at[pl.ds(r, _RC)], g0)
            pltpu.sync_copy(g0, xt.at[pl.ds(xoff + r, _RC)])
            _zero_acc(r)
            pltpu.sync_copy(out_ref.at[pl.ds(xoff + r, _RC)], g1)

            @pl.loop(0, _RC, unroll=8)
            def _(i):
                a0 = (g1[i, pl.ds(0, 16)] + g0[i, pl.ds(0, 16)]) * sc16
                a1 = (g1[i, pl.ds(16, 16)] + g0[i, pl.ds(16, 16)]) * sc16
                g1[i, pl.ds(0, 16)] = a0
                g1[i, pl.ds(16, 16)] = a1

            pltpu.sync_copy(g1, out_ref.at[pl.ds(xoff + r, _RC)])

        plsc.subcore_barrier()


@jax.jit
def _sc_call(row_p, col_p, val_p, x0il):
    mesh = plsc.VectorSubcoreMesh(core_axis_name="c", subcore_axis_name="s")
    f = pl.kernel(
        _sc_body,
        out_type=jax.ShapeDtypeStruct((2 * _NP, _H), jnp.float32),
        mesh=mesh,
        compiler_params=pltpu.CompilerParams(use_tc_tiling_on_sc=False),
        scratch_types=[
            pltpu.HBM((2 * _NP, _H), jnp.float32),
            pltpu.VMEM_SHARED((_NP, _H), jnp.float32),
            pltpu.VMEM((_C, _H), jnp.float32),
            pltpu.VMEM((_C, _H), jnp.float32),
            pltpu.VMEM((_C, _H), jnp.float32),
            pltpu.VMEM((_C, _H), jnp.float32),
            pltpu.VMEM((32, _H), jnp.float32),
            pltpu.VMEM((_BLK,), jnp.int32),
            pltpu.VMEM((_NCPB, _C), jnp.int32),
            pltpu.VMEM((_BLK,), jnp.float32),
            pltpu.VMEM((_BLK,), jnp.int32),
            pltpu.VMEM((_NCPB, _C), jnp.int32),
            pltpu.VMEM((_BLK,), jnp.float32),
            pltpu.SemaphoreType.DMA,
            pltpu.SemaphoreType.DMA,
            pltpu.SemaphoreType.DMA,
            pltpu.SemaphoreType.DMA,
            pltpu.SemaphoreType.DMA,
            pltpu.SemaphoreType.DMA,
            pltpu.SemaphoreType.DMA,
            pltpu.SemaphoreType.DMA,
            pltpu.SemaphoreType.DMA,
            pltpu.SemaphoreType.DMA,
        ],
    )
    return f(row_p, col_p, val_p, x0il)


def kernel(adj_indices, adj_values, user_emb_w, item_emb_w):
    x0 = jnp.concatenate([user_emb_w, item_emb_w], axis=0)
    x0il = x0.reshape(2 * _N, _H)
    pad = _E_PAD - _E
    pidx = jnp.arange(pad, dtype=jnp.int32) % _N
    row_p = jnp.concatenate([adj_indices[0].astype(jnp.int32), pidx])
    col_p = jnp.concatenate([adj_indices[1].astype(jnp.int32), pidx])
    val_p = jnp.concatenate([adj_values, jnp.zeros((pad,), jnp.float32)])
    row2d = row_p.reshape(_E_PAD // _C, _C)
    outil = _sc_call(row2d, col_p, val_p, x0il)
    halves = outil.reshape(2, _NP, _H)
    mean = jnp.concatenate([halves[0, :_N], halves[1, :_N]], axis=1)
    return mean[:_NUM_USER], mean[_NUM_USER:]
